# baseline (speedup 1.0000x reference)
import functools

import jax
import jax.numpy as jnp
from jax import lax
from jax.experimental import pallas as pl
from jax.experimental.pallas import tpu as pltpu
from jax.experimental.pallas import tpu_sc as plsc

B = 16384
D = 64
NEG = 20
L = 16
NVR = D // L

NC = 2
NS = 16
NW = NC * NS
BPW = B // NW
C = 64
NCH = BPW // C

_mesh = plsc.VectorSubcoreMesh(core_axis_name="c", subcore_axis_name="s")


@functools.partial(
    pl.kernel,
    mesh=_mesh,
    compiler_params=pltpu.CompilerParams(use_tc_tiling_on_sc=False),
    out_type=(
        jax.ShapeDtypeStruct((B, L), jnp.float32),
        jax.ShapeDtypeStruct((B, L), jnp.float32),
    ),
    scratch_types=[
        pltpu.VMEM((C,), jnp.int32),
        pltpu.VMEM((C,), jnp.int32),
        pltpu.VMEM((NEG * C,), jnp.int32),
        pltpu.VMEM((C, D), jnp.float32),
        pltpu.VMEM((C, D), jnp.float32),
        pltpu.VMEM((NEG * C, D), jnp.float32),
        pltpu.VMEM((C, L), jnp.float32),
        pltpu.VMEM((C, L), jnp.float32),
        pltpu.SemaphoreType.DMA,
        pltpu.SemaphoreType.DMA,
        pltpu.SemaphoreType.DMA,
    ],
)
def _sc_partials(cidx_hbm, kidx_hbm, nidx_hbm, iemb_hbm, oemb_hbm,
                 posp_hbm, negp_hbm,
                 cidx_v, kidx_v, nidx_v, crow_v, krow_v, nrow_v,
                 posp_v, negp_v, sem_c, sem_k, sem_n):
    wid = lax.axis_index("s") * NC + lax.axis_index("c")
    for ch in range(NCH):
        base = wid * BPW + ch * C
        pltpu.sync_copy(cidx_hbm.at[pl.ds(base, C)], cidx_v)
        pltpu.sync_copy(kidx_hbm.at[pl.ds(base, C)], kidx_v)
        pltpu.sync_copy(nidx_hbm.at[pl.ds(base * NEG, C * NEG)], nidx_v)
        cp_c = pltpu.async_copy(iemb_hbm.at[cidx_v], crow_v, sem_c)
        cp_k = pltpu.async_copy(oemb_hbm.at[kidx_v], krow_v, sem_k)
        cp_n = pltpu.async_copy(oemb_hbm.at[nidx_v], nrow_v, sem_n)
        cp_c.wait()
        cp_k.wait()
        cp_n.wait()

        def row_body(r, carry):
            c = [crow_v[r, pl.ds(L * v, L)] for v in range(NVR)]
            k = [krow_v[r, pl.ds(L * v, L)] for v in range(NVR)]
            pp = c[0] * k[0] + c[1] * k[1] + c[2] * k[2] + c[3] * k[3]
            posp_v[r, :] = pp
            nbase = r * NEG
            s = [nrow_v[nbase, pl.ds(L * v, L)] for v in range(NVR)]
            for n in range(1, NEG):
                for v in range(NVR):
                    s[v] = s[v] + nrow_v[nbase + n, pl.ds(L * v, L)]
            np_ = c[0] * s[0] + c[1] * s[1] + c[2] * s[2] + c[3] * s[3]
            negp_v[r, :] = np_
            return carry

        lax.fori_loop(0, C, row_body, 0)
        pltpu.sync_copy(posp_v, posp_hbm.at[pl.ds(base, C)])
        pltpu.sync_copy(negp_v, negp_hbm.at[pl.ds(base, C)])


def _tc_loss_body(pp_ref, np_ref, out_ref):
    sp = jnp.sum(pp_ref[...], axis=1, keepdims=True)
    sn = jnp.sum(np_ref[...], axis=1, keepdims=True)

    def logsig(x):
        return jnp.minimum(x, 0.0) - jnp.log(1.0 + jnp.exp(-jnp.abs(x)))

    tot = jnp.sum(logsig(sp) + logsig(-sn))
    out_ref[...] = jnp.full((1, 1), -(tot / B), dtype=jnp.float32)


_tc_loss = pl.pallas_call(
    _tc_loss_body,
    out_shape=jax.ShapeDtypeStruct((1, 1), jnp.float32),
)


def kernel(center_words, context_words, neg_context_words,
           input_embeddings, output_embeddings):
    cidx = center_words.astype(jnp.int32)
    kidx = context_words.astype(jnp.int32)
    nidx = neg_context_words.astype(jnp.int32).reshape(B * NEG)
    posp, negp = _sc_partials(cidx, kidx, nidx,
                              input_embeddings, output_embeddings)
    return _tc_loss(posp, negp)[0, 0]


# speedup vs baseline: 1.0324x; 1.0324x over previous
import functools

import jax
import jax.numpy as jnp
from jax import lax
from jax.experimental import pallas as pl
from jax.experimental.pallas import tpu as pltpu
from jax.experimental.pallas import tpu_sc as plsc

B = 16384
D = 64
NEG = 20
L = 16
NVR = D // L

NC = 2
NS = 16
NW = NC * NS
C = B // NW

_mesh = plsc.VectorSubcoreMesh(core_axis_name="c", subcore_axis_name="s")


@functools.partial(
    pl.kernel,
    mesh=_mesh,
    compiler_params=pltpu.CompilerParams(use_tc_tiling_on_sc=False),
    out_type=(
        jax.ShapeDtypeStruct((B, L), jnp.float32),
        jax.ShapeDtypeStruct((B, L), jnp.float32),
    ),
    scratch_types=[
        pltpu.VMEM((C,), jnp.int32),
        pltpu.VMEM((C,), jnp.int32),
        pltpu.VMEM((NEG, C), jnp.int32),
        pltpu.VMEM((C, D), jnp.float32),
        pltpu.VMEM((C, D), jnp.float32),
        pltpu.VMEM((C, D), jnp.float32),
        pltpu.VMEM((C, L), jnp.float32),
        pltpu.VMEM((C, L), jnp.float32),
        pltpu.SemaphoreType.DMA,
        pltpu.SemaphoreType.DMA,
        pltpu.SemaphoreType.DMA,
        pltpu.SemaphoreType.DMA,
    ],
)
def _sc_partials(cidx_hbm, kidx_hbm, nidx_hbm, iemb_hbm, oemb_hbm,
                 posp_hbm, negp_hbm,
                 cidx_v, kidx_v, nidx_v, crow_v, krow_v, acc_v,
                 posp_v, negp_v, sem_c, sem_k, sem_i, sem_n):
    wid = lax.axis_index("s") * NC + lax.axis_index("c")
    base = wid * C
    pltpu.sync_copy(cidx_hbm.at[pl.ds(base, C)], cidx_v)
    pltpu.sync_copy(kidx_hbm.at[pl.ds(base, C)], kidx_v)
    cp_c = pltpu.async_copy(iemb_hbm.at[cidx_v], crow_v, sem_c)
    cp_k = pltpu.async_copy(oemb_hbm.at[kidx_v], krow_v, sem_k)
    idx_cps = [
        pltpu.async_copy(nidx_hbm.at[n, pl.ds(base, C)], nidx_v.at[n], sem_i)
        for n in range(NEG)
    ]

    def zero_body(r, carry):
        for v in range(NVR):
            acc_v[r, pl.ds(L * v, L)] = jnp.zeros((L,), jnp.float32)
        return carry

    lax.fori_loop(0, C, zero_body, 0)
    for cp in idx_cps:
        cp.wait()

    for n in range(NEG):
        pltpu.async_copy(oemb_hbm.at[nidx_v.at[n]], acc_v, sem_n, add=True)
    for n in range(NEG):
        pltpu.make_async_copy(oemb_hbm.at[nidx_v.at[0]], acc_v, sem_n).wait()
    cp_c.wait()
    cp_k.wait()

    def row_body(r, carry):
        c = [crow_v[r, pl.ds(L * v, L)] for v in range(NVR)]
        k = [krow_v[r, pl.ds(L * v, L)] for v in range(NVR)]
        a = [acc_v[r, pl.ds(L * v, L)] for v in range(NVR)]
        pp = c[0] * k[0] + c[1] * k[1] + c[2] * k[2] + c[3] * k[3]
        nn = c[0] * a[0] + c[1] * a[1] + c[2] * a[2] + c[3] * a[3]
        posp_v[r, :] = pp
        negp_v[r, :] = nn
        return carry

    lax.fori_loop(0, C, row_body, 0)
    pltpu.sync_copy(posp_v, posp_hbm.at[pl.ds(base, C)])
    pltpu.sync_copy(negp_v, negp_hbm.at[pl.ds(base, C)])


def _tc_loss_body(pp_ref, np_ref, out_ref):
    sp = jnp.sum(pp_ref[...], axis=1, keepdims=True)
    sn = jnp.sum(np_ref[...], axis=1, keepdims=True)

    def logsig(x):
        return jnp.minimum(x, 0.0) - jnp.log(1.0 + jnp.exp(-jnp.abs(x)))

    tot = jnp.sum(logsig(sp) + logsig(-sn))
    out_ref[...] = jnp.full((1, 1), -(tot / B), dtype=jnp.float32)


_tc_loss = pl.pallas_call(
    _tc_loss_body,
    out_shape=jax.ShapeDtypeStruct((1, 1), jnp.float32),
)


def kernel(center_words, context_words, neg_context_words,
           input_embeddings, output_embeddings):
    cidx = center_words.astype(jnp.int32)
    kidx = context_words.astype(jnp.int32)
    izero = lax.optimization_barrier(jnp.int32(0))
    nidx = neg_context_words.astype(jnp.int32).T + izero
    posp, negp = _sc_partials(cidx, kidx, nidx,
                              input_embeddings, output_embeddings)
    fzero = lax.optimization_barrier(jnp.float32(0))
    return _tc_loss(posp + fzero, negp + fzero)[0, 0]


# speedup vs baseline: 1.0606x; 1.0273x over previous
import functools

import jax
import jax.numpy as jnp
from jax import lax
from jax.experimental import pallas as pl
from jax.experimental.pallas import tpu as pltpu
from jax.experimental.pallas import tpu_sc as plsc

B = 16384
D = 64
NEG = 20
L = 16
NVR = D // L

NC = 2
NS = 16
NW = NC * NS
BPW = B // NW
C = 256
NCH = BPW // C
DP = 128

_mesh = plsc.VectorSubcoreMesh(core_axis_name="c", subcore_axis_name="s")


@functools.partial(
    pl.kernel,
    mesh=_mesh,
    compiler_params=pltpu.CompilerParams(use_tc_tiling_on_sc=False),
    out_type=(
        jax.ShapeDtypeStruct((B, L), jnp.float32),
        jax.ShapeDtypeStruct((B, L), jnp.float32),
    ),
    scratch_types=[
        pltpu.VMEM((C,), jnp.int32),
        pltpu.VMEM((C,), jnp.int32),
        pltpu.VMEM((NEG, C), jnp.int32),
        pltpu.VMEM((C, DP), jnp.float32),
        pltpu.VMEM((C, DP), jnp.float32),
        pltpu.VMEM((C, DP), jnp.float32),
        pltpu.VMEM((C, L), jnp.float32),
        pltpu.VMEM((C, L), jnp.float32),
        pltpu.SemaphoreType.DMA,
        pltpu.SemaphoreType.DMA,
        pltpu.SemaphoreType.DMA,
        pltpu.SemaphoreType.DMA,
    ],
)
def _sc_partials(cidx_hbm, kidx_hbm, nidx_hbm, iemb_hbm, oemb_hbm,
                 posp_hbm, negp_hbm,
                 cidx_v, kidx_v, nidx_v, crow_v, krow_v, acc_v,
                 posp_v, negp_v, sem_c, sem_k, sem_i, sem_n):
    wid = lax.axis_index("s") * NC + lax.axis_index("c")
    for ch in range(NCH):
        base = wid * BPW + ch * C
        pltpu.sync_copy(cidx_hbm.at[pl.ds(base, C)], cidx_v)
        pltpu.sync_copy(kidx_hbm.at[pl.ds(base, C)], kidx_v)
        cp_c = pltpu.async_copy(iemb_hbm.at[cidx_v], crow_v, sem_c)
        cp_k = pltpu.async_copy(oemb_hbm.at[kidx_v], krow_v, sem_k)
        idx_cps = [
            pltpu.async_copy(nidx_hbm.at[n, pl.ds(base, C)], nidx_v.at[n],
                             sem_i)
            for n in range(NEG)
        ]

        def zero_body(r, carry):
            for v in range(NVR):
                acc_v[r, pl.ds(L * v, L)] = jnp.zeros((L,), jnp.float32)
            return carry

        lax.fori_loop(0, C, zero_body, 0)
        for cp in idx_cps:
            cp.wait()

        for n in range(NEG):
            pltpu.async_copy(oemb_hbm.at[nidx_v.at[n]], acc_v, sem_n,
                             add=True)
        for n in range(NEG):
            pltpu.make_async_copy(oemb_hbm.at[nidx_v.at[0]], acc_v,
                                  sem_n).wait()
        cp_c.wait()
        cp_k.wait()

        def row_body(r, carry):
            c = [crow_v[r, pl.ds(L * v, L)] for v in range(NVR)]
            k = [krow_v[r, pl.ds(L * v, L)] for v in range(NVR)]
            a = [acc_v[r, pl.ds(L * v, L)] for v in range(NVR)]
            pp = c[0] * k[0] + c[1] * k[1] + c[2] * k[2] + c[3] * k[3]
            nn = c[0] * a[0] + c[1] * a[1] + c[2] * a[2] + c[3] * a[3]
            posp_v[r, :] = pp
            negp_v[r, :] = nn
            return carry

        lax.fori_loop(0, C, row_body, 0)
        pltpu.sync_copy(posp_v, posp_hbm.at[pl.ds(base, C)])
        pltpu.sync_copy(negp_v, negp_hbm.at[pl.ds(base, C)])


def _tc_loss_body(pp_ref, np_ref, out_ref):
    sp = jnp.sum(pp_ref[...], axis=1, keepdims=True)
    sn = jnp.sum(np_ref[...], axis=1, keepdims=True)

    def logsig(x):
        return jnp.minimum(x, 0.0) - jnp.log(1.0 + jnp.exp(-jnp.abs(x)))

    tot = jnp.sum(logsig(sp) + logsig(-sn))
    out_ref[...] = jnp.full((1, 1), -(tot / B), dtype=jnp.float32)


_tc_loss = pl.pallas_call(
    _tc_loss_body,
    out_shape=jax.ShapeDtypeStruct((1, 1), jnp.float32),
)


def kernel(center_words, context_words, neg_context_words,
           input_embeddings, output_embeddings):
    cidx = center_words.astype(jnp.int32)
    kidx = context_words.astype(jnp.int32)
    izero = lax.optimization_barrier(jnp.int32(0))
    nidx = neg_context_words.astype(jnp.int32).T + izero
    iemb = jnp.pad(input_embeddings, ((0, 0), (0, DP - D)))
    oemb = jnp.pad(output_embeddings, ((0, 0), (0, DP - D)))
    posp, negp = _sc_partials(cidx, kidx, nidx, iemb, oemb)
    fzero = lax.optimization_barrier(jnp.float32(0))
    return _tc_loss(posp + fzero, negp + fzero)[0, 0]
